# baseline (speedup 1.0000x reference)
import functools

import jax
import jax.numpy as jnp
from jax import lax
from jax.experimental import pallas as pl
from jax.experimental.pallas import tpu as pltpu
from jax.experimental.pallas import tpu_sc as plsc

N_NODES = 10000
E_EDGES = 320000
U_DIM = 128
EPS_BN = 1e-5

NC = 2
NS = 16
NW = NC * NS
EPT = E_EDGES // NW
BB = 128
NB = 80
EPTB = NB * BB
ROWS_PT = N_NODES // NS
_ROW_CHUNKS = ((0, 128), (128, 128), (256, 128), (384, 128), (512, 113))

_MESH = plsc.VectorSubcoreMesh(core_axis_name="c", subcore_axis_name="s")
_SC_PARAMS = pltpu.CompilerParams(needs_layout_passes=False,
                                  use_tc_tiling_on_sc=False)


def _deg_body(src_hbm, dst_hbm, w_hbm, out_hbm, src_v, dst_v, w_v, acc_v):
    c = lax.axis_index("c")
    s = lax.axis_index("s")
    wid = s * NC + c
    ebase = wid * EPT
    z16 = jnp.zeros((16,), jnp.float32)

    def zero_step(i, carry):
        acc_v[pl.ds(i * 16, 16)] = z16
        return carry

    lax.fori_loop(0, N_NODES // 16, zero_step, 0)
    pltpu.sync_copy(src_hbm.at[pl.ds(ebase, EPT)], src_v)
    pltpu.sync_copy(dst_hbm.at[pl.ds(ebase, EPT)], dst_v)
    pltpu.sync_copy(w_hbm.at[pl.ds(ebase, EPT)], w_v)

    def step(i, carry):
        sl = pl.ds(i * 16, 16)
        sv = src_v[sl]
        wv = jnp.where(sv == dst_v[sl], 0.0, w_v[sl])
        plsc.addupdate_scatter(acc_v, [sv], wv)
        return carry

    lax.fori_loop(0, EPT // 16, step, 0)
    pltpu.sync_copy(acc_v, out_hbm.at[wid])


_sc_deg = functools.partial(
    pl.kernel,
    out_type=jax.ShapeDtypeStruct((NW, N_NODES), jnp.float32),
    mesh=_MESH,
    compiler_params=_SC_PARAMS,
    scratch_types=[
        pltpu.VMEM((EPT,), jnp.int32),
        pltpu.VMEM((EPT,), jnp.int32),
        pltpu.VMEM((EPT,), jnp.float32),
        pltpu.VMEM((N_NODES,), jnp.float32),
    ],
)(_deg_body)


def _normw_body(src_hbm, dst_hbm, w_hbm, dinv_hbm, out_hbm,
                src_v, dst_v, w_v, dinv_v, nw_v):
    c = lax.axis_index("c")
    s = lax.axis_index("s")
    wid = s * NC + c
    ebase = wid * EPT
    pltpu.sync_copy(dinv_hbm, dinv_v)
    pltpu.sync_copy(src_hbm.at[pl.ds(ebase, EPT)], src_v)
    pltpu.sync_copy(dst_hbm.at[pl.ds(ebase, EPT)], dst_v)
    pltpu.sync_copy(w_hbm.at[pl.ds(ebase, EPT)], w_v)

    def step(i, carry):
        sl = pl.ds(i * 16, 16)
        sv = src_v[sl]
        dv = dst_v[sl]
        wv = jnp.where(sv == dv, 0.0, w_v[sl])
        da = plsc.load_gather(dinv_v, [sv])
        db = plsc.load_gather(dinv_v, [dv])
        nw_v[sl] = -(da * wv * db)
        return carry

    lax.fori_loop(0, EPT // 16, step, 0)
    pltpu.sync_copy(nw_v, out_hbm.at[pl.ds(ebase, EPT)])


_sc_normw = functools.partial(
    pl.kernel,
    out_type=jax.ShapeDtypeStruct((E_EDGES,), jnp.float32),
    mesh=_MESH,
    compiler_params=_SC_PARAMS,
    scratch_types=[
        pltpu.VMEM((EPT,), jnp.int32),
        pltpu.VMEM((EPT,), jnp.int32),
        pltpu.VMEM((EPT,), jnp.float32),
        pltpu.VMEM((N_NODES,), jnp.float32),
        pltpu.VMEM((EPT,), jnp.float32),
    ],
)(_normw_body)


def _lap_body(v_hbm, src_hbm, dst_hbm, nw_hbm, out_hbm,
              acc_sh, src_a, src_b, dst_a, dst_b, nw_a, nw_b,
              rows_a, rows_b, sem_a, sem_b):
    c = lax.axis_index("c")
    s = lax.axis_index("s")
    wid = s * NC + c
    rbase = s * ROWS_PT
    z16 = jnp.zeros((16,), jnp.float32)

    def zrow(r, carry):
        for j in range(U_DIM // 16):
            rows_a[r, pl.ds(j * 16, 16)] = z16
        return carry

    lax.fori_loop(0, BB, zrow, 0)
    for off, ln in _ROW_CHUNKS:
        pltpu.sync_copy(rows_a.at[pl.ds(0, ln)], acc_sh.at[pl.ds(rbase + off, ln)])
    plsc.subcore_barrier()

    def fetch(b, sbuf, dbuf, nbuf):
        off = pl.ds(b * BB, BB)
        pltpu.sync_copy(src_hbm.at[wid, off], sbuf)
        pltpu.sync_copy(dst_hbm.at[wid, off], dbuf)
        pltpu.sync_copy(nw_hbm.at[wid, off], nbuf)

    def g_start(sbuf, rows, sem):
        pltpu.make_async_copy(v_hbm.at[sbuf], rows, sem).start()

    def g_wait(sbuf, rows, sem):
        pltpu.make_async_copy(v_hbm.at[sbuf], rows, sem).wait()

    def process(rows, nbuf, dbuf):
        def scale_group(g, carry):
            nw16 = nbuf[pl.ds(g * 16, 16)]
            for k in range(16):
                w = nw16[k]
                r = g * 16 + k
                for j in range(U_DIM // 16):
                    sl = pl.ds(j * 16, 16)
                    rows[r, sl] = rows[r, sl] * w
            return carry

        lax.fori_loop(0, BB // 16, scale_group, 0)
        pltpu.sync_copy(rows, acc_sh.at[dbuf], add=True)

    def body(b, carry):
        fetch(b, src_a, dst_a, nw_a)
        g_start(src_a, rows_a, sem_a)
        g_wait(src_a, rows_a, sem_a)
        process(rows_a, nw_a, dst_a)
        return carry

    lax.fori_loop(0, NB, body, 0)

    plsc.subcore_barrier()
    pltpu.sync_copy(acc_sh.at[pl.ds(rbase, ROWS_PT)],
                    out_hbm.at[c, pl.ds(rbase, ROWS_PT)])


_sc_lap = functools.partial(
    pl.kernel,
    out_type=jax.ShapeDtypeStruct((NC, N_NODES, U_DIM), jnp.float32),
    mesh=_MESH,
    compiler_params=_SC_PARAMS,
    scratch_types=[
        pltpu.VMEM_SHARED((N_NODES, U_DIM), jnp.float32),
        pltpu.VMEM((BB,), jnp.int32),
        pltpu.VMEM((BB,), jnp.int32),
        pltpu.VMEM((BB,), jnp.int32),
        pltpu.VMEM((BB,), jnp.int32),
        pltpu.VMEM((BB,), jnp.float32),
        pltpu.VMEM((BB,), jnp.float32),
        pltpu.VMEM((BB, U_DIM), jnp.float32),
        pltpu.VMEM((BB, U_DIM), jnp.float32),
        pltpu.SemaphoreType.DMA,
        pltpu.SemaphoreType.DMA,
    ],
)(_lap_body)


def _dinv_body(parts_ref, out_ref):
    deg = jnp.sum(parts_ref[...], axis=0)
    out_ref[...] = jnp.where(deg > 0, lax.rsqrt(deg), 0.0)


_tc_dinv = pl.pallas_call(
    _dinv_body,
    out_shape=jax.ShapeDtypeStruct((N_NODES,), jnp.float32),
)


def _sum2_body(p_ref, o_ref):
    o_ref[...] = p_ref[0] + p_ref[1]


_tc_sum2 = pl.pallas_call(
    _sum2_body,
    grid=(5,),
    in_specs=[pl.BlockSpec((2, N_NODES // 5, U_DIM), lambda i: (0, i, 0))],
    out_specs=pl.BlockSpec((N_NODES // 5, U_DIM), lambda i: (i, 0)),
    out_shape=jax.ShapeDtypeStruct((N_NODES, U_DIM), jnp.float32),
)


def _combine_body(h_ref, tx1_ref, l2p_ref, w_ref, b_ref, g_ref, beta_ref, o_ref):
    h = h_ref[...]
    tx1 = tx1_ref[...]
    tx2 = 2.0 * (l2p_ref[0] + l2p_ref[1]) - h
    sacc = (jnp.dot(h, w_ref[0], preferred_element_type=jnp.float32)
            + jnp.dot(tx1, w_ref[1], preferred_element_type=jnp.float32)
            + jnp.dot(tx2, w_ref[2], preferred_element_type=jnp.float32)
            + b_ref[...])
    mu = jnp.mean(sacc, axis=0, keepdims=True)
    var = jnp.mean((sacc - mu) ** 2, axis=0, keepdims=True)
    y = (sacc - mu) * lax.rsqrt(var + EPS_BN) * g_ref[...] + beta_ref[...]
    o_ref[...] = jnp.maximum(y, 0.0)


_tc_combine = pl.pallas_call(
    _combine_body,
    out_shape=jax.ShapeDtypeStruct((N_NODES, U_DIM), jnp.float32),
)


def _final_body(h_ref, nw_ref, gw_ref, nb_ref, gb_ref, ln_ref, lg_ref):
    h = h_ref[...]
    ln_ref[...] = jnp.dot(h, nw_ref[...], preferred_element_type=jnp.float32) + nb_ref[0, 0]
    lg_ref[...] = (jnp.sum(h * gw_ref[...]) + gb_ref[0, 0])[None, None]


_tc_final = pl.pallas_call(
    _final_body,
    out_shape=(
        jax.ShapeDtypeStruct((N_NODES, 1), jnp.float32),
        jax.ShapeDtypeStruct((1, 1), jnp.float32),
    ),
)


def kernel(x, edge_index, weights, batch, params):
    del batch
    src = edge_index[0]
    dst = edge_index[1]

    deg_parts = _sc_deg(src, dst, weights)
    dinv = _tc_dinv(deg_parts)
    normw = _sc_normw(src, dst, weights, dinv)

    pad = ((0, 0), (0, EPTB - EPT))
    src2 = jnp.pad(src.reshape(NW, EPT), pad)
    pad_dst = (jnp.arange(EPTB - EPT, dtype=jnp.int32)[None, :]
               + (EPTB - EPT) * jnp.arange(NW, dtype=jnp.int32)[:, None]) % N_NODES
    dst2 = jnp.concatenate([dst.reshape(NW, EPT), pad_dst], axis=1)
    nw2 = jnp.pad(normw.reshape(NW, EPT), pad)

    h = x
    for l in range(5):
        l1p = _sc_lap(h, src2, dst2, nw2)
        tx1 = _tc_sum2(l1p)
        l2p = _sc_lap(tx1, src2, dst2, nw2)
        h = _tc_combine(h, tx1, l2p, params[f"W{l}"],
                        params[f"b{l}"].reshape(1, U_DIM),
                        params[f"g{l}"].reshape(1, U_DIM),
                        params[f"beta{l}"].reshape(1, U_DIM))

    ln, lg = _tc_final(h,
                       params["node_w"].reshape(U_DIM, 1),
                       params["graph_w"].reshape(N_NODES, U_DIM),
                       params["node_b"].reshape(1, 1),
                       params["graph_b"].reshape(1, 1))
    logits_nodes = ln.reshape(1, N_NODES)
    logits_graph = lg.reshape(1,)
    return logits_nodes, logits_graph


# speedup vs baseline: 2.4838x; 2.4838x over previous
import functools

import jax
import jax.numpy as jnp
from jax import lax
from jax.experimental import pallas as pl
from jax.experimental.pallas import tpu as pltpu
from jax.experimental.pallas import tpu_sc as plsc

N_NODES = 10000
E_EDGES = 320000
U_DIM = 128
EPS_BN = 1e-5

NC = 2
NS = 16
NW = NC * NS
EPT = E_EDGES // NW
BB = 128
NB = 80
EPTB = NB * BB
ROWS_PT = N_NODES // NS
_ROW_CHUNKS = ((0, 128), (128, 128), (256, 128), (384, 128), (512, 113))

_MESH = plsc.VectorSubcoreMesh(core_axis_name="c", subcore_axis_name="s")
_SC_PARAMS = pltpu.CompilerParams(needs_layout_passes=False,
                                  use_tc_tiling_on_sc=False)


def _deg_body(src_hbm, dst_hbm, w_hbm, out_hbm, src_v, dst_v, w_v, acc_v):
    c = lax.axis_index("c")
    s = lax.axis_index("s")
    wid = s * NC + c
    ebase = wid * EPT
    z16 = jnp.zeros((16,), jnp.float32)

    def zero_step(i, carry):
        acc_v[pl.ds(i * 16, 16)] = z16
        return carry

    lax.fori_loop(0, N_NODES // 16, zero_step, 0)
    pltpu.sync_copy(src_hbm.at[pl.ds(ebase, EPT)], src_v)
    pltpu.sync_copy(dst_hbm.at[pl.ds(ebase, EPT)], dst_v)
    pltpu.sync_copy(w_hbm.at[pl.ds(ebase, EPT)], w_v)

    def step(i, carry):
        sl = pl.ds(i * 16, 16)
        sv = src_v[sl]
        wv = jnp.where(sv == dst_v[sl], 0.0, w_v[sl])
        plsc.addupdate_scatter(acc_v, [sv], wv)
        return carry

    lax.fori_loop(0, EPT // 16, step, 0)
    pltpu.sync_copy(acc_v, out_hbm.at[wid])


_sc_deg = functools.partial(
    pl.kernel,
    out_type=jax.ShapeDtypeStruct((NW, N_NODES), jnp.float32),
    mesh=_MESH,
    compiler_params=_SC_PARAMS,
    scratch_types=[
        pltpu.VMEM((EPT,), jnp.int32),
        pltpu.VMEM((EPT,), jnp.int32),
        pltpu.VMEM((EPT,), jnp.float32),
        pltpu.VMEM((N_NODES,), jnp.float32),
    ],
)(_deg_body)


def _normw_body(src_hbm, dst_hbm, w_hbm, dinv_hbm, out_hbm,
                src_v, dst_v, w_v, dinv_v, nw_v):
    c = lax.axis_index("c")
    s = lax.axis_index("s")
    wid = s * NC + c
    ebase = wid * EPT
    pltpu.sync_copy(dinv_hbm, dinv_v)
    pltpu.sync_copy(src_hbm.at[pl.ds(ebase, EPT)], src_v)
    pltpu.sync_copy(dst_hbm.at[pl.ds(ebase, EPT)], dst_v)
    pltpu.sync_copy(w_hbm.at[pl.ds(ebase, EPT)], w_v)

    def step(i, carry):
        sl = pl.ds(i * 16, 16)
        sv = src_v[sl]
        dv = dst_v[sl]
        wv = jnp.where(sv == dv, 0.0, w_v[sl])
        da = plsc.load_gather(dinv_v, [sv])
        db = plsc.load_gather(dinv_v, [dv])
        nw_v[sl] = -(da * wv * db)
        return carry

    lax.fori_loop(0, EPT // 16, step, 0)
    pltpu.sync_copy(nw_v, out_hbm.at[pl.ds(ebase, EPT)])


_sc_normw = functools.partial(
    pl.kernel,
    out_type=jax.ShapeDtypeStruct((E_EDGES,), jnp.float32),
    mesh=_MESH,
    compiler_params=_SC_PARAMS,
    scratch_types=[
        pltpu.VMEM((EPT,), jnp.int32),
        pltpu.VMEM((EPT,), jnp.int32),
        pltpu.VMEM((EPT,), jnp.float32),
        pltpu.VMEM((N_NODES,), jnp.float32),
        pltpu.VMEM((EPT,), jnp.float32),
    ],
)(_normw_body)


NFULL = EPT // BB
TAIL = EPT - NFULL * BB


def _lap_body(v_hbm, src_hbm, dst_hbm, nw_hbm, out_hbm,
              acc_sh, src_a, src_b, dst_a, dst_b, tsrc, tdst, nw_a, nw_b,
              rows_a, rows_b, sem_a, sem_b):
    c = lax.axis_index("c")
    s = lax.axis_index("s")
    wid = s * NC + c
    ebase = wid * EPT
    rbase = s * ROWS_PT
    z16 = jnp.zeros((16,), jnp.float32)

    def zrow(r, carry):
        for j in range(U_DIM // 16):
            rows_a[r, pl.ds(j * 16, 16)] = z16
        return carry

    lax.fori_loop(0, BB, zrow, 0)
    for off, ln in _ROW_CHUNKS:
        pltpu.sync_copy(rows_a.at[pl.ds(0, ln)], acc_sh.at[pl.ds(rbase + off, ln)])
    plsc.subcore_barrier()

    def fetch(bstart, sbuf, dbuf, nbuf):
        pltpu.sync_copy(src_hbm.at[pl.ds(bstart, BB)], sbuf)
        pltpu.sync_copy(dst_hbm.at[pl.ds(bstart, BB)], dbuf)
        pltpu.sync_copy(nw_hbm.at[pl.ds(bstart, BB)], nbuf)

    def scale(blen, rows, nbuf):
        def scale_group(g, carry):
            nw16 = nbuf[pl.ds(g * 16, 16)]
            for k in range(16):
                w = nw16[k]
                r = g * 16 + k
                for j in range(U_DIM // 16):
                    sl = pl.ds(j * 16, 16)
                    rows[r, sl] = rows[r, sl] * w
            return carry

        lax.fori_loop(0, blen // 16, scale_group, 0)

    def body(i, carry):
        b0 = ebase + 2 * i * BB
        fetch(b0, src_a, dst_a, nw_a)
        ha = pltpu.async_copy(v_hbm.at[src_a], rows_a, sem_a)
        fetch(b0 + BB, src_b, dst_b, nw_b)
        hb = pltpu.async_copy(v_hbm.at[src_b], rows_b, sem_b)
        ha.wait()
        scale(BB, rows_a, nw_a)
        pltpu.sync_copy(rows_a, acc_sh.at[dst_a], add=True)
        hb.wait()
        scale(BB, rows_b, nw_b)
        pltpu.sync_copy(rows_b, acc_sh.at[dst_b], add=True)
        return carry

    lax.fori_loop(0, NFULL // 2, body, 0)

    tb = ebase + NFULL * BB
    pltpu.sync_copy(src_hbm.at[pl.ds(tb, TAIL)], tsrc)
    pltpu.sync_copy(dst_hbm.at[pl.ds(tb, TAIL)], tdst)
    pltpu.sync_copy(nw_hbm.at[pl.ds(tb, TAIL)], nw_a.at[pl.ds(0, TAIL)])
    pltpu.async_copy(v_hbm.at[tsrc], rows_a.at[pl.ds(0, TAIL)], sem_a).wait()
    scale(TAIL, rows_a, nw_a)
    pltpu.sync_copy(rows_a.at[pl.ds(0, TAIL)], acc_sh.at[tdst], add=True)

    plsc.subcore_barrier()
    pltpu.sync_copy(acc_sh.at[pl.ds(rbase, ROWS_PT)],
                    out_hbm.at[c, pl.ds(rbase, ROWS_PT)])


_sc_lap = functools.partial(
    pl.kernel,
    out_type=jax.ShapeDtypeStruct((NC, N_NODES, U_DIM), jnp.float32),
    mesh=_MESH,
    compiler_params=_SC_PARAMS,
    scratch_types=[
        pltpu.VMEM_SHARED((N_NODES, U_DIM), jnp.float32),
        pltpu.VMEM((BB,), jnp.int32),
        pltpu.VMEM((BB,), jnp.int32),
        pltpu.VMEM((BB,), jnp.int32),
        pltpu.VMEM((BB,), jnp.int32),
        pltpu.VMEM((TAIL,), jnp.int32),
        pltpu.VMEM((TAIL,), jnp.int32),
        pltpu.VMEM((BB,), jnp.float32),
        pltpu.VMEM((BB,), jnp.float32),
        pltpu.VMEM((BB, U_DIM), jnp.float32),
        pltpu.VMEM((BB, U_DIM), jnp.float32),
        pltpu.SemaphoreType.DMA,
        pltpu.SemaphoreType.DMA,
    ],
)(_lap_body)


def _dinv_body(parts_ref, out_ref):
    deg = jnp.sum(parts_ref[...], axis=0)
    out_ref[...] = jnp.where(deg > 0, lax.rsqrt(deg), 0.0)


_tc_dinv = pl.pallas_call(
    _dinv_body,
    out_shape=jax.ShapeDtypeStruct((N_NODES,), jnp.float32),
)


def _sum2_body(p_ref, o_ref):
    o_ref[...] = p_ref[0] + p_ref[1]


_tc_sum2 = pl.pallas_call(
    _sum2_body,
    grid=(5,),
    in_specs=[pl.BlockSpec((2, N_NODES // 5, U_DIM), lambda i: (0, i, 0))],
    out_specs=pl.BlockSpec((N_NODES // 5, U_DIM), lambda i: (i, 0)),
    out_shape=jax.ShapeDtypeStruct((N_NODES, U_DIM), jnp.float32),
)


def _combine_body(h_ref, tx1_ref, l2p_ref, w_ref, b_ref, g_ref, beta_ref, o_ref):
    h = h_ref[...]
    tx1 = tx1_ref[...]
    tx2 = 2.0 * (l2p_ref[0] + l2p_ref[1]) - h
    sacc = (jnp.dot(h, w_ref[0], preferred_element_type=jnp.float32)
            + jnp.dot(tx1, w_ref[1], preferred_element_type=jnp.float32)
            + jnp.dot(tx2, w_ref[2], preferred_element_type=jnp.float32)
            + b_ref[...])
    mu = jnp.mean(sacc, axis=0, keepdims=True)
    var = jnp.mean((sacc - mu) ** 2, axis=0, keepdims=True)
    y = (sacc - mu) * lax.rsqrt(var + EPS_BN) * g_ref[...] + beta_ref[...]
    o_ref[...] = jnp.maximum(y, 0.0)


_tc_combine = pl.pallas_call(
    _combine_body,
    out_shape=jax.ShapeDtypeStruct((N_NODES, U_DIM), jnp.float32),
)


def _final_body(h_ref, nw_ref, gw_ref, nb_ref, gb_ref, ln_ref, lg_ref):
    h = h_ref[...]
    ln_ref[...] = jnp.dot(h, nw_ref[...], preferred_element_type=jnp.float32) + nb_ref[0, 0]
    lg_ref[...] = (jnp.sum(h * gw_ref[...]) + gb_ref[0, 0])[None, None]


_tc_final = pl.pallas_call(
    _final_body,
    out_shape=(
        jax.ShapeDtypeStruct((N_NODES, 1), jnp.float32),
        jax.ShapeDtypeStruct((1, 1), jnp.float32),
    ),
)


def kernel(x, edge_index, weights, batch, params):
    del batch
    src = edge_index[0]
    dst = edge_index[1]

    deg_parts = _sc_deg(src, dst, weights)
    dinv = _tc_dinv(deg_parts)
    normw = _sc_normw(src, dst, weights, dinv)

    h = x
    for l in range(5):
        l1p = _sc_lap(h, src, dst, normw)
        tx1 = _tc_sum2(l1p)
        l2p = _sc_lap(tx1, src, dst, normw)
        h = _tc_combine(h, tx1, l2p, params[f"W{l}"],
                        params[f"b{l}"].reshape(1, U_DIM),
                        params[f"g{l}"].reshape(1, U_DIM),
                        params[f"beta{l}"].reshape(1, U_DIM))

    ln, lg = _tc_final(h,
                       params["node_w"].reshape(U_DIM, 1),
                       params["graph_w"].reshape(N_NODES, U_DIM),
                       params["node_b"].reshape(1, 1),
                       params["graph_b"].reshape(1, 1))
    logits_nodes = ln.reshape(1, N_NODES)
    logits_graph = lg.reshape(1,)
    return logits_nodes, logits_graph


# speedup vs baseline: 2.8753x; 1.1576x over previous
import functools

import jax
import jax.numpy as jnp
from jax import lax
from jax.experimental import pallas as pl
from jax.experimental.pallas import tpu as pltpu
from jax.experimental.pallas import tpu_sc as plsc

N_NODES = 10000
E_EDGES = 320000
U_DIM = 128
EPS_BN = 1e-5

NC = 2
NS = 16
NW = NC * NS
EPT = E_EDGES // NW
BB = 128
NB = 80
EPTB = NB * BB
ROWS_PT = N_NODES // NS
_ROW_CHUNKS = ((0, 128), (128, 128), (256, 128), (384, 128), (512, 113))

_MESH = plsc.VectorSubcoreMesh(core_axis_name="c", subcore_axis_name="s")
_SC_PARAMS = pltpu.CompilerParams(needs_layout_passes=False,
                                  use_tc_tiling_on_sc=False)


def _deg_body(src_hbm, dst_hbm, w_hbm, out_hbm, src_v, dst_v, w_v, acc_v):
    c = lax.axis_index("c")
    s = lax.axis_index("s")
    wid = s * NC + c
    ebase = wid * EPT
    z16 = jnp.zeros((16,), jnp.float32)

    def zero_step(i, carry):
        acc_v[pl.ds(i * 16, 16)] = z16
        return carry

    lax.fori_loop(0, N_NODES // 16, zero_step, 0)
    pltpu.sync_copy(src_hbm.at[pl.ds(ebase, EPT)], src_v)
    pltpu.sync_copy(dst_hbm.at[pl.ds(ebase, EPT)], dst_v)
    pltpu.sync_copy(w_hbm.at[pl.ds(ebase, EPT)], w_v)

    def step(i, carry):
        sl = pl.ds(i * 16, 16)
        sv = src_v[sl]
        wv = jnp.where(sv == dst_v[sl], 0.0, w_v[sl])
        plsc.addupdate_scatter(acc_v, [sv], wv)
        return carry

    lax.fori_loop(0, EPT // 16, step, 0)
    pltpu.sync_copy(acc_v, out_hbm.at[wid])


_sc_deg = functools.partial(
    pl.kernel,
    out_type=jax.ShapeDtypeStruct((NW, N_NODES), jnp.float32),
    mesh=_MESH,
    compiler_params=_SC_PARAMS,
    scratch_types=[
        pltpu.VMEM((EPT,), jnp.int32),
        pltpu.VMEM((EPT,), jnp.int32),
        pltpu.VMEM((EPT,), jnp.float32),
        pltpu.VMEM((N_NODES,), jnp.float32),
    ],
)(_deg_body)


def _normw_body(src_hbm, dst_hbm, w_hbm, dinv_hbm, out_hbm,
                src_v, dst_v, w_v, dinv_v, nw_v):
    c = lax.axis_index("c")
    s = lax.axis_index("s")
    wid = s * NC + c
    ebase = wid * EPT
    pltpu.sync_copy(dinv_hbm, dinv_v)
    pltpu.sync_copy(src_hbm.at[pl.ds(ebase, EPT)], src_v)
    pltpu.sync_copy(dst_hbm.at[pl.ds(ebase, EPT)], dst_v)
    pltpu.sync_copy(w_hbm.at[pl.ds(ebase, EPT)], w_v)

    def step(i, carry):
        sl = pl.ds(i * 16, 16)
        sv = src_v[sl]
        dv = dst_v[sl]
        wv = jnp.where(sv == dv, 0.0, w_v[sl])
        da = plsc.load_gather(dinv_v, [sv])
        db = plsc.load_gather(dinv_v, [dv])
        nw_v[sl] = -(da * wv * db)
        return carry

    lax.fori_loop(0, EPT // 16, step, 0)
    pltpu.sync_copy(nw_v, out_hbm.at[pl.ds(ebase, EPT)])


_sc_normw = functools.partial(
    pl.kernel,
    out_type=jax.ShapeDtypeStruct((E_EDGES,), jnp.float32),
    mesh=_MESH,
    compiler_params=_SC_PARAMS,
    scratch_types=[
        pltpu.VMEM((EPT,), jnp.int32),
        pltpu.VMEM((EPT,), jnp.int32),
        pltpu.VMEM((EPT,), jnp.float32),
        pltpu.VMEM((N_NODES,), jnp.float32),
        pltpu.VMEM((EPT,), jnp.float32),
    ],
)(_normw_body)


NFULL = EPT // BB
TAIL = EPT - NFULL * BB
NBODY = (NFULL - 2) // 4


def _lap_body(v_hbm, src_hbm, dst_hbm, nw_hbm, out_hbm,
              acc_sh,
              s00, d00, n00, s01, d01, n01,
              s10, d10, n10, s11, d11, n11,
              tsrc, tdst,
              rows_a, rows_b, sem_a, sem_b, sem_i0, sem_i1):
    c = lax.axis_index("c")
    s = lax.axis_index("s")
    wid = s * NC + c
    ebase = wid * EPT
    rbase = s * ROWS_PT
    z16 = jnp.zeros((16,), jnp.float32)

    def zrow(r, carry):
        for j in range(U_DIM // 16):
            rows_a[r, pl.ds(j * 16, 16)] = z16
        return carry

    lax.fori_loop(0, BB, zrow, 0)
    for off, ln in _ROW_CHUNKS:
        pltpu.sync_copy(rows_a.at[pl.ds(0, ln)], acc_sh.at[pl.ds(rbase + off, ln)])
    plsc.subcore_barrier()

    def pair_copies(b0, bufs, sem):
        sb0, db0, nb0, sb1, db1, nb1 = bufs
        o0 = pl.ds(ebase + b0 * BB, BB)
        o1 = pl.ds(ebase + (b0 + 1) * BB, BB)
        return (
            pltpu.make_async_copy(src_hbm.at[o0], sb0, sem),
            pltpu.make_async_copy(dst_hbm.at[o0], db0, sem),
            pltpu.make_async_copy(nw_hbm.at[o0], nb0, sem),
            pltpu.make_async_copy(src_hbm.at[o1], sb1, sem),
            pltpu.make_async_copy(dst_hbm.at[o1], db1, sem),
            pltpu.make_async_copy(nw_hbm.at[o1], nb1, sem),
        )

    def fetch_pair_start(b0, bufs, sem):
        for h in pair_copies(b0, bufs, sem):
            h.start()

    def fetch_pair_wait(b0, bufs, sem):
        for h in pair_copies(b0, bufs, sem):
            h.wait()

    def scale(blen, rows, nbuf):
        def scale_group(g, carry):
            nw16 = nbuf[pl.ds(g * 16, 16)]
            for k in range(16):
                w = nw16[k]
                r = g * 16 + k
                for j in range(U_DIM // 16):
                    sl = pl.ds(j * 16, 16)
                    rows[r, sl] = rows[r, sl] * w
            return carry

        lax.fori_loop(0, blen // 16, scale_group, 0)

    def run_pair(bufs):
        sb0, db0, nb0, sb1, db1, nb1 = bufs
        h0 = pltpu.async_copy(v_hbm.at[sb0], rows_a, sem_a)
        h1 = pltpu.async_copy(v_hbm.at[sb1], rows_b, sem_b)
        h0.wait()
        scale(BB, rows_a, nb0)
        pltpu.sync_copy(rows_a, acc_sh.at[db0], add=True)
        h1.wait()
        scale(BB, rows_b, nb1)
        pltpu.sync_copy(rows_b, acc_sh.at[db1], add=True)

    bufs0 = (s00, d00, n00, s01, d01, n01)
    bufs1 = (s10, d10, n10, s11, d11, n11)

    fetch_pair_start(0, bufs0, sem_i0)
    fetch_pair_wait(0, bufs0, sem_i0)
    fetch_pair_start(2, bufs1, sem_i1)

    def body(j, carry):
        b0 = 4 * j
        run_pair(bufs0)
        fetch_pair_wait(b0 + 2, bufs1, sem_i1)
        fetch_pair_start(b0 + 4, bufs0, sem_i0)
        run_pair(bufs1)
        fetch_pair_wait(b0 + 4, bufs0, sem_i0)
        fetch_pair_start(b0 + 6, bufs1, sem_i1)
        return carry

    lax.fori_loop(0, NBODY, body, 0)

    run_pair(bufs0)
    fetch_pair_wait(NFULL, bufs1, sem_i1)

    tb = ebase + NFULL * BB
    pltpu.sync_copy(src_hbm.at[pl.ds(tb, TAIL)], tsrc)
    pltpu.sync_copy(dst_hbm.at[pl.ds(tb, TAIL)], tdst)
    pltpu.sync_copy(nw_hbm.at[pl.ds(tb, TAIL)], n00.at[pl.ds(0, TAIL)])
    pltpu.async_copy(v_hbm.at[tsrc], rows_a.at[pl.ds(0, TAIL)], sem_a).wait()
    scale(TAIL, rows_a, n00)
    pltpu.sync_copy(rows_a.at[pl.ds(0, TAIL)], acc_sh.at[tdst], add=True)

    plsc.subcore_barrier()
    pltpu.sync_copy(acc_sh.at[pl.ds(rbase, ROWS_PT)],
                    out_hbm.at[c, pl.ds(rbase, ROWS_PT)])


_idx_scratch = [pltpu.VMEM((BB,), jnp.int32)] * 2 + [pltpu.VMEM((BB,), jnp.float32)]

_sc_lap = functools.partial(
    pl.kernel,
    out_type=jax.ShapeDtypeStruct((NC, N_NODES, U_DIM), jnp.float32),
    mesh=_MESH,
    compiler_params=_SC_PARAMS,
    scratch_types=(
        [pltpu.VMEM_SHARED((N_NODES, U_DIM), jnp.float32)]
        + _idx_scratch * 4
        + [pltpu.VMEM((TAIL,), jnp.int32)] * 2
        + [
            pltpu.VMEM((BB, U_DIM), jnp.float32),
            pltpu.VMEM((BB, U_DIM), jnp.float32),
            pltpu.SemaphoreType.DMA,
            pltpu.SemaphoreType.DMA,
            pltpu.SemaphoreType.DMA,
            pltpu.SemaphoreType.DMA,
        ]
    ),
)(_lap_body)


def _dinv_body(parts_ref, out_ref):
    deg = jnp.sum(parts_ref[...], axis=0)
    out_ref[...] = jnp.where(deg > 0, lax.rsqrt(deg), 0.0)


_tc_dinv = pl.pallas_call(
    _dinv_body,
    out_shape=jax.ShapeDtypeStruct((N_NODES,), jnp.float32),
)


def _sum2_body(p_ref, o_ref):
    o_ref[...] = p_ref[0] + p_ref[1]


_tc_sum2 = pl.pallas_call(
    _sum2_body,
    grid=(5,),
    in_specs=[pl.BlockSpec((2, N_NODES // 5, U_DIM), lambda i: (0, i, 0))],
    out_specs=pl.BlockSpec((N_NODES // 5, U_DIM), lambda i: (i, 0)),
    out_shape=jax.ShapeDtypeStruct((N_NODES, U_DIM), jnp.float32),
)


def _combine_body(h_ref, tx1_ref, l2p_ref, w_ref, b_ref, g_ref, beta_ref, o_ref):
    h = h_ref[...]
    tx1 = tx1_ref[...]
    tx2 = 2.0 * (l2p_ref[0] + l2p_ref[1]) - h
    sacc = (jnp.dot(h, w_ref[0], preferred_element_type=jnp.float32)
            + jnp.dot(tx1, w_ref[1], preferred_element_type=jnp.float32)
            + jnp.dot(tx2, w_ref[2], preferred_element_type=jnp.float32)
            + b_ref[...])
    mu = jnp.mean(sacc, axis=0, keepdims=True)
    var = jnp.mean((sacc - mu) ** 2, axis=0, keepdims=True)
    y = (sacc - mu) * lax.rsqrt(var + EPS_BN) * g_ref[...] + beta_ref[...]
    o_ref[...] = jnp.maximum(y, 0.0)


_tc_combine = pl.pallas_call(
    _combine_body,
    out_shape=jax.ShapeDtypeStruct((N_NODES, U_DIM), jnp.float32),
)


def _final_body(h_ref, nw_ref, gw_ref, nb_ref, gb_ref, ln_ref, lg_ref):
    h = h_ref[...]
    ln_ref[...] = jnp.dot(h, nw_ref[...], preferred_element_type=jnp.float32) + nb_ref[0, 0]
    lg_ref[...] = (jnp.sum(h * gw_ref[...]) + gb_ref[0, 0])[None, None]


_tc_final = pl.pallas_call(
    _final_body,
    out_shape=(
        jax.ShapeDtypeStruct((N_NODES, 1), jnp.float32),
        jax.ShapeDtypeStruct((1, 1), jnp.float32),
    ),
)


def kernel(x, edge_index, weights, batch, params):
    del batch
    src = edge_index[0]
    dst = edge_index[1]

    deg_parts = _sc_deg(src, dst, weights)
    dinv = _tc_dinv(deg_parts)
    normw = _sc_normw(src, dst, weights, dinv)

    srcp = jnp.pad(src, (0, 2 * BB))
    dstp = jnp.pad(dst, (0, 2 * BB))
    nwp = jnp.pad(normw, (0, 2 * BB))

    h = x
    for l in range(5):
        l1p = _sc_lap(h, srcp, dstp, nwp)
        tx1 = _tc_sum2(l1p)
        l2p = _sc_lap(tx1, srcp, dstp, nwp)
        h = _tc_combine(h, tx1, l2p, params[f"W{l}"],
                        params[f"b{l}"].reshape(1, U_DIM),
                        params[f"g{l}"].reshape(1, U_DIM),
                        params[f"beta{l}"].reshape(1, U_DIM))

    ln, lg = _tc_final(h,
                       params["node_w"].reshape(U_DIM, 1),
                       params["graph_w"].reshape(N_NODES, U_DIM),
                       params["node_b"].reshape(1, 1),
                       params["graph_b"].reshape(1, 1))
    logits_nodes = ln.reshape(1, N_NODES)
    logits_graph = lg.reshape(1,)
    return logits_nodes, logits_graph


# speedup vs baseline: 3.6038x; 1.2534x over previous
import functools

import jax
import jax.numpy as jnp
from jax import lax
from jax.experimental import pallas as pl
from jax.experimental.pallas import tpu as pltpu
from jax.experimental.pallas import tpu_sc as plsc

N_NODES = 10000
E_EDGES = 320000
U_DIM = 128
EPS_BN = 1e-5

NC = 2
NS = 16
NW = NC * NS
EPT = E_EDGES // NW
BB = 128
NB = 80
EPTB = NB * BB
ROWS_PT = N_NODES // NS
_ROW_CHUNKS = ((0, 128), (128, 128), (256, 128), (384, 128), (512, 113))

_MESH = plsc.VectorSubcoreMesh(core_axis_name="c", subcore_axis_name="s")
_SC_PARAMS = pltpu.CompilerParams(needs_layout_passes=False,
                                  use_tc_tiling_on_sc=False)


def _deg_body(src_hbm, dst_hbm, w_hbm, out_hbm, src_v, dst_v, w_v, acc_v):
    c = lax.axis_index("c")
    s = lax.axis_index("s")
    wid = s * NC + c
    ebase = wid * EPT
    z16 = jnp.zeros((16,), jnp.float32)

    def zero_step(i, carry):
        acc_v[pl.ds(i * 16, 16)] = z16
        return carry

    lax.fori_loop(0, N_NODES // 16, zero_step, 0)
    pltpu.sync_copy(src_hbm.at[pl.ds(ebase, EPT)], src_v)
    pltpu.sync_copy(dst_hbm.at[pl.ds(ebase, EPT)], dst_v)
    pltpu.sync_copy(w_hbm.at[pl.ds(ebase, EPT)], w_v)

    def step(i, carry):
        sl = pl.ds(i * 16, 16)
        sv = src_v[sl]
        wv = jnp.where(sv == dst_v[sl], 0.0, w_v[sl])
        plsc.addupdate_scatter(acc_v, [sv], wv)
        return carry

    lax.fori_loop(0, EPT // 16, step, 0)
    pltpu.sync_copy(acc_v, out_hbm.at[wid])


_sc_deg = functools.partial(
    pl.kernel,
    out_type=jax.ShapeDtypeStruct((NW, N_NODES), jnp.float32),
    mesh=_MESH,
    compiler_params=_SC_PARAMS,
    scratch_types=[
        pltpu.VMEM((EPT,), jnp.int32),
        pltpu.VMEM((EPT,), jnp.int32),
        pltpu.VMEM((EPT,), jnp.float32),
        pltpu.VMEM((N_NODES,), jnp.float32),
    ],
)(_deg_body)


def _normw_body(src_hbm, dst_hbm, w_hbm, dinv_hbm, out_hbm,
                src_v, dst_v, w_v, dinv_v, nw_v):
    c = lax.axis_index("c")
    s = lax.axis_index("s")
    wid = s * NC + c
    ebase = wid * EPT
    pltpu.sync_copy(dinv_hbm, dinv_v)
    pltpu.sync_copy(src_hbm.at[pl.ds(ebase, EPT)], src_v)
    pltpu.sync_copy(dst_hbm.at[pl.ds(ebase, EPT)], dst_v)
    pltpu.sync_copy(w_hbm.at[pl.ds(ebase, EPT)], w_v)

    def step(i, carry):
        sl = pl.ds(i * 16, 16)
        sv = src_v[sl]
        dv = dst_v[sl]
        wv = jnp.where(sv == dv, 0.0, w_v[sl])
        da = plsc.load_gather(dinv_v, [sv])
        db = plsc.load_gather(dinv_v, [dv])
        nw_v[sl] = -(da * wv * db)
        return carry

    lax.fori_loop(0, EPT // 16, step, 0)
    pltpu.sync_copy(nw_v, out_hbm.at[pl.ds(ebase, EPT)])


_sc_normw = functools.partial(
    pl.kernel,
    out_type=jax.ShapeDtypeStruct((E_EDGES,), jnp.float32),
    mesh=_MESH,
    compiler_params=_SC_PARAMS,
    scratch_types=[
        pltpu.VMEM((EPT,), jnp.int32),
        pltpu.VMEM((EPT,), jnp.int32),
        pltpu.VMEM((EPT,), jnp.float32),
        pltpu.VMEM((N_NODES,), jnp.float32),
        pltpu.VMEM((EPT,), jnp.float32),
    ],
)(_normw_body)


NFULL = EPT // BB
TAIL = EPT - NFULL * BB
NBODY = (NFULL // 2 - 1) // 2


def _lap_body(v_hbm, src_hbm, dst_hbm, nw_hbm, out_hbm,
              acc_sh,
              s00, d00, n00, s01, d01, n01,
              s10, d10, n10, s11, d11, n11,
              tsrc, tdst,
              rows_a, rows_b, sem_a, sem_b, sem_i0, sem_i1, sem_sa, sem_sb):
    c = lax.axis_index("c")
    s = lax.axis_index("s")
    wid = s * NC + c
    ebase = wid * EPT
    rbase = s * ROWS_PT
    z16 = jnp.zeros((16,), jnp.float32)

    def zrow(r, carry):
        for j in range(U_DIM // 16):
            rows_a[r, pl.ds(j * 16, 16)] = z16
            rows_b[r, pl.ds(j * 16, 16)] = z16
        return carry

    lax.fori_loop(0, BB, zrow, 0)
    for off, ln in _ROW_CHUNKS:
        pltpu.sync_copy(rows_a.at[pl.ds(0, ln)], acc_sh.at[pl.ds(rbase + off, ln)])
    plsc.subcore_barrier()

    def pair_copies(b0, bufs, sem):
        sb0, db0, nb0, sb1, db1, nb1 = bufs
        o0 = pl.ds(ebase + b0 * BB, BB)
        o1 = pl.ds(ebase + (b0 + 1) * BB, BB)
        return (
            pltpu.make_async_copy(src_hbm.at[o0], sb0, sem),
            pltpu.make_async_copy(dst_hbm.at[o0], db0, sem),
            pltpu.make_async_copy(nw_hbm.at[o0], nb0, sem),
            pltpu.make_async_copy(src_hbm.at[o1], sb1, sem),
            pltpu.make_async_copy(dst_hbm.at[o1], db1, sem),
            pltpu.make_async_copy(nw_hbm.at[o1], nb1, sem),
        )

    def fetch_pair_start(b0, bufs, sem):
        for h in pair_copies(b0, bufs, sem):
            h.start()

    def fetch_pair_wait(b0, bufs, sem):
        for h in pair_copies(b0, bufs, sem):
            h.wait()

    def scale(blen, rows, nbuf):
        def scale_group(g, carry):
            nw16 = nbuf[pl.ds(g * 16, 16)]
            for k in range(16):
                w = nw16[k]
                r = g * 16 + k
                for j in range(U_DIM // 16):
                    sl = pl.ds(j * 16, 16)
                    rows[r, sl] = rows[r, sl] * w
            return carry

        lax.fori_loop(0, blen // 16, scale_group, 0)

    def wait_scatters(db0, db1):
        pltpu.make_async_copy(rows_a, acc_sh.at[db0], sem_sa).wait()
        pltpu.make_async_copy(rows_b, acc_sh.at[db1], sem_sb).wait()

    def run_pair(bufs, prev_db0, prev_db1, refetch=None):
        sb0, db0, nb0, sb1, db1, nb1 = bufs
        pltpu.make_async_copy(rows_a, acc_sh.at[prev_db0], sem_sa).wait()
        h0 = pltpu.async_copy(v_hbm.at[sb0], rows_a, sem_a)
        pltpu.make_async_copy(rows_b, acc_sh.at[prev_db1], sem_sb).wait()
        h1 = pltpu.async_copy(v_hbm.at[sb1], rows_b, sem_b)
        if refetch is not None:
            rb, rbufs, rsem = refetch
            fetch_pair_start(rb, rbufs, rsem)
        h0.wait()
        scale(BB, rows_a, nb0)
        pltpu.async_copy(rows_a, acc_sh.at[db0], sem_sa, add=True)
        h1.wait()
        scale(BB, rows_b, nb1)
        pltpu.async_copy(rows_b, acc_sh.at[db1], sem_sb, add=True)
        if refetch is not None:
            rb, rbufs, rsem = refetch
            fetch_pair_wait(rb, rbufs, rsem)

    bufs0 = (s00, d00, n00, s01, d01, n01)
    bufs1 = (s10, d10, n10, s11, d11, n11)

    fetch_pair_start(0, bufs0, sem_i0)
    fetch_pair_start(2, bufs1, sem_i1)
    fetch_pair_wait(0, bufs0, sem_i0)
    fetch_pair_wait(2, bufs1, sem_i1)
    pltpu.async_copy(rows_a, acc_sh.at[d10], sem_sa, add=True)
    pltpu.async_copy(rows_b, acc_sh.at[d11], sem_sb, add=True)

    run_pair(bufs0, d10, d11)

    def body(j, carry):
        b0 = 4 * j
        run_pair(bufs1, d00, d01, refetch=(b0 + 4, bufs0, sem_i0))
        run_pair(bufs0, d10, d11, refetch=(b0 + 6, bufs1, sem_i1))
        return carry

    lax.fori_loop(0, NBODY, body, 0)

    wait_scatters(d00, d01)

    tb = ebase + NFULL * BB
    pltpu.sync_copy(src_hbm.at[pl.ds(tb, TAIL)], tsrc)
    pltpu.sync_copy(dst_hbm.at[pl.ds(tb, TAIL)], tdst)
    pltpu.sync_copy(nw_hbm.at[pl.ds(tb, TAIL)], n00.at[pl.ds(0, TAIL)])
    pltpu.async_copy(v_hbm.at[tsrc], rows_a.at[pl.ds(0, TAIL)], sem_a).wait()
    scale(TAIL, rows_a, n00)
    pltpu.sync_copy(rows_a.at[pl.ds(0, TAIL)], acc_sh.at[tdst], add=True)

    plsc.subcore_barrier()
    pltpu.sync_copy(acc_sh.at[pl.ds(rbase, ROWS_PT)],
                    out_hbm.at[c, pl.ds(rbase, ROWS_PT)])


_idx_scratch = [pltpu.VMEM((BB,), jnp.int32)] * 2 + [pltpu.VMEM((BB,), jnp.float32)]

_sc_lap = functools.partial(
    pl.kernel,
    out_type=jax.ShapeDtypeStruct((NC, N_NODES, U_DIM), jnp.float32),
    mesh=_MESH,
    compiler_params=_SC_PARAMS,
    scratch_types=(
        [pltpu.VMEM_SHARED((N_NODES, U_DIM), jnp.float32)]
        + _idx_scratch * 4
        + [pltpu.VMEM((TAIL,), jnp.int32)] * 2
        + [
            pltpu.VMEM((BB, U_DIM), jnp.float32),
            pltpu.VMEM((BB, U_DIM), jnp.float32),
        ]
        + [pltpu.SemaphoreType.DMA] * 6
    ),
)(_lap_body)


def _dinv_body(parts_ref, out_ref):
    deg = jnp.sum(parts_ref[...], axis=0)
    out_ref[...] = jnp.where(deg > 0, lax.rsqrt(deg), 0.0)


_tc_dinv = pl.pallas_call(
    _dinv_body,
    out_shape=jax.ShapeDtypeStruct((N_NODES,), jnp.float32),
)


def _sum2_body(p_ref, o_ref):
    o_ref[...] = p_ref[0] + p_ref[1]


_tc_sum2 = pl.pallas_call(
    _sum2_body,
    grid=(5,),
    in_specs=[pl.BlockSpec((2, N_NODES // 5, U_DIM), lambda i: (0, i, 0))],
    out_specs=pl.BlockSpec((N_NODES // 5, U_DIM), lambda i: (i, 0)),
    out_shape=jax.ShapeDtypeStruct((N_NODES, U_DIM), jnp.float32),
)


def _combine_body(h_ref, tx1_ref, l2p_ref, w_ref, b_ref, g_ref, beta_ref, o_ref):
    h = h_ref[...]
    tx1 = tx1_ref[...]
    tx2 = 2.0 * (l2p_ref[0] + l2p_ref[1]) - h
    sacc = (jnp.dot(h, w_ref[0], preferred_element_type=jnp.float32)
            + jnp.dot(tx1, w_ref[1], preferred_element_type=jnp.float32)
            + jnp.dot(tx2, w_ref[2], preferred_element_type=jnp.float32)
            + b_ref[...])
    mu = jnp.mean(sacc, axis=0, keepdims=True)
    var = jnp.mean((sacc - mu) ** 2, axis=0, keepdims=True)
    y = (sacc - mu) * lax.rsqrt(var + EPS_BN) * g_ref[...] + beta_ref[...]
    o_ref[...] = jnp.maximum(y, 0.0)


_tc_combine = pl.pallas_call(
    _combine_body,
    out_shape=jax.ShapeDtypeStruct((N_NODES, U_DIM), jnp.float32),
)


def _final_body(h_ref, nw_ref, gw_ref, nb_ref, gb_ref, ln_ref, lg_ref):
    h = h_ref[...]
    ln_ref[...] = jnp.dot(h, nw_ref[...], preferred_element_type=jnp.float32) + nb_ref[0, 0]
    lg_ref[...] = (jnp.sum(h * gw_ref[...]) + gb_ref[0, 0])[None, None]


_tc_final = pl.pallas_call(
    _final_body,
    out_shape=(
        jax.ShapeDtypeStruct((N_NODES, 1), jnp.float32),
        jax.ShapeDtypeStruct((1, 1), jnp.float32),
    ),
)


def kernel(x, edge_index, weights, batch, params):
    del batch
    src = edge_index[0]
    dst = edge_index[1]

    deg_parts = _sc_deg(src, dst, weights)
    dinv = _tc_dinv(deg_parts)
    normw = _sc_normw(src, dst, weights, dinv)

    srcp = jnp.pad(src, (0, 2 * BB))
    dstp = jnp.pad(dst, (0, 2 * BB))
    nwp = jnp.pad(normw, (0, 2 * BB))

    h = x
    for l in range(5):
        l1p = _sc_lap(h, srcp, dstp, nwp)
        tx1 = _tc_sum2(l1p)
        l2p = _sc_lap(tx1, srcp, dstp, nwp)
        h = _tc_combine(h, tx1, l2p, params[f"W{l}"],
                        params[f"b{l}"].reshape(1, U_DIM),
                        params[f"g{l}"].reshape(1, U_DIM),
                        params[f"beta{l}"].reshape(1, U_DIM))

    ln, lg = _tc_final(h,
                       params["node_w"].reshape(U_DIM, 1),
                       params["graph_w"].reshape(N_NODES, U_DIM),
                       params["node_b"].reshape(1, 1),
                       params["graph_b"].reshape(1, 1))
    logits_nodes = ln.reshape(1, N_NODES)
    logits_graph = lg.reshape(1,)
    return logits_nodes, logits_graph
